# Initial kernel scaffold; baseline (speedup 1.0000x reference)
import functools

import jax
import jax.numpy as jnp
from jax import lax
from jax.experimental import pallas as pl
from jax.experimental.pallas import tpu as pltpu
from jax.experimental.pallas import tpu_sc as plsc

NN = 100000
NE = 1600000
LN = 128
EROWS = NE // LN
NCORE, NSUB = 2, 16
NW = NCORE * NSUB
RB = 4000
GRID = NN // RB
CP = 6248
CPT = NN - NSUB * CP

_MESH = plsc.VectorSubcoreMesh(core_axis_name="c", subcore_axis_name="s")


def _fill(ref, n, val):
    def body(i, carry):
        ref[pl.ds(i * 16, 16)] = jnp.full((16,), val, dtype=jnp.float32)
        return carry
    lax.fori_loop(0, n // 16, body, 0)


def _zero_acc(zeros_hbm, acc, s):
    pltpu.sync_copy(zeros_hbm.at[pl.ds(0, CP)], acc.at[pl.ds(s * CP, CP)])

    @pl.when(s == NSUB - 1)
    def _():
        pltpu.sync_copy(zeros_hbm.at[pl.ds(0, CPT)],
                        acc.at[pl.ds(NSUB * CP, CPT)])


def _copy_out(acc, out, s):
    pltpu.sync_copy(acc.at[pl.ds(s * CP, CP)], out.at[pl.ds(s * CP, CP)])

    @pl.when(s == NSUB - 1)
    def _():
        pltpu.sync_copy(acc.at[pl.ds(NSUB * CP, CPT)],
                        out.at[pl.ds(NSUB * CP, CPT)])


def _edge_range_all32(w):
    rem = EROWS - 390 * NW
    n = jnp.where(w < rem, 391, 390)
    r0 = 390 * w + jnp.minimum(w, rem)
    return r0, n


def _edge_range_16(s):
    rem = EROWS - 781 * NSUB
    n = jnp.where(s < rem, 782, 781)
    r0 = 781 * s + jnp.minimum(s, rem)
    return r0, n


@functools.partial(
    pl.kernel,
    out_type=(jax.ShapeDtypeStruct((NN,), jnp.float32),
              jax.ShapeDtypeStruct((NN,), jnp.float32)),
    mesh=_MESH,
    scratch_types=[
        pltpu.VMEM((1, LN), jnp.int32),
        pltpu.VMEM((LN,), jnp.float32),
        pltpu.VMEM_SHARED((NN,), jnp.float32),
    ],
)
def _sc_deg(dst_hbm, zeros_hbm, out0, out1, idx_d, ones_v, acc):
    c = lax.axis_index("c")
    s = lax.axis_index("s")
    w = c * NSUB + s
    _zero_acc(zeros_hbm, acc, s)
    _fill(ones_v, LN, 1.0)
    plsc.subcore_barrier()

    r0, n = _edge_range_all32(w)

    def body(r, carry):
        pltpu.sync_copy(dst_hbm.at[pl.ds(r, 1)], idx_d)
        pltpu.sync_copy(ones_v, acc.at[idx_d.at[0]], add=True)
        return carry

    lax.fori_loop(r0, r0 + n, body, 0)
    plsc.subcore_barrier()

    @pl.when(c == 0)
    def _():
        _copy_out(acc, out0, s)

    @pl.when(c == 1)
    def _():
        _copy_out(acc, out1, s)


@functools.partial(
    pl.kernel,
    out_type=(jax.ShapeDtypeStruct((NN, 2), jnp.float32),
              jax.ShapeDtypeStruct((NN, 2), jnp.float32)),
    mesh=_MESH,
    scratch_types=[
        pltpu.VMEM((1, LN), jnp.int32),
        pltpu.VMEM((1, LN), jnp.int32),
        pltpu.VMEM((LN, 2), jnp.float32),
        pltpu.VMEM_SHARED((NN, 2), jnp.float32),
        pltpu.SemaphoreType.DMA,
    ],
)
def _sc_agg1(src_hbm, dst_hbm, xs_hbm, zeros_hbm, out0, out1,
             idx_s, idx_d, rows, acc, sem):
    c = lax.axis_index("c")
    s = lax.axis_index("s")
    w = c * NSUB + s
    _zero_acc(zeros_hbm, acc, s)
    plsc.subcore_barrier()

    r0, n = _edge_range_all32(w)

    def body(r, carry):
        pltpu.sync_copy(src_hbm.at[pl.ds(r, 1)], idx_s)
        pltpu.sync_copy(dst_hbm.at[pl.ds(r, 1)], idx_d)
        pltpu.async_copy(xs_hbm.at[idx_s.at[0]], rows, sem).wait()
        pltpu.sync_copy(rows, acc.at[idx_d.at[0]], add=True)
        return carry

    lax.fori_loop(r0, r0 + n, body, 0)
    plsc.subcore_barrier()

    @pl.when(c == 0)
    def _():
        _copy_out(acc, out0, s)

    @pl.when(c == 1)
    def _():
        _copy_out(acc, out1, s)


@functools.partial(
    pl.kernel,
    out_type=(jax.ShapeDtypeStruct((NN, 16), jnp.float32),
              jax.ShapeDtypeStruct((NN, 16), jnp.float32)),
    mesh=_MESH,
    scratch_types=[
        pltpu.VMEM((1, LN), jnp.int32),
        pltpu.VMEM((1, LN), jnp.int32),
        pltpu.VMEM((LN, 16), jnp.float32),
        pltpu.VMEM_SHARED((NN, 16), jnp.float32),
        pltpu.SemaphoreType.DMA,
    ],
)
def _sc_agg2(src_hbm, dst_hbm, ha_hbm, hb_hbm, zeros_hbm, outa, outb,
             idx_s, idx_d, rows, acc, sem):
    c = lax.axis_index("c")
    s = lax.axis_index("s")
    _zero_acc(zeros_hbm, acc, s)
    plsc.subcore_barrier()

    r0, n = _edge_range_16(s)

    def body(r, carry):
        pltpu.sync_copy(src_hbm.at[pl.ds(r, 1)], idx_s)
        pltpu.sync_copy(dst_hbm.at[pl.ds(r, 1)], idx_d)

        @pl.when(c == 0)
        def _():
            pltpu.async_copy(ha_hbm.at[idx_s.at[0]], rows, sem).wait()

        @pl.when(c == 1)
        def _():
            pltpu.async_copy(hb_hbm.at[idx_s.at[0]], rows, sem).wait()

        pltpu.sync_copy(rows, acc.at[idx_d.at[0]], add=True)
        return carry

    lax.fori_loop(r0, r0 + n, body, 0)
    plsc.subcore_barrier()

    @pl.when(c == 0)
    def _():
        _copy_out(acc, outa, s)

    @pl.when(c == 1)
    def _():
        _copy_out(acc, outb, s)


def _tc_prep(x, deg0, deg1):
    def body(x_ref, d0_ref, d1_ref, dinv_ref, xs_ref):
        deg = d0_ref[...] + d1_ref[...] + 1.0
        dinv = lax.rsqrt(deg)
        dinv_ref[...] = dinv
        xs_ref[...] = x_ref[...] * dinv

    return pl.pallas_call(
        body,
        grid=(GRID,),
        in_specs=[pl.BlockSpec((RB, 2), lambda i: (i, 0)),
                  pl.BlockSpec((RB, 1), lambda i: (i, 0)),
                  pl.BlockSpec((RB, 1), lambda i: (i, 0))],
        out_specs=[pl.BlockSpec((RB, 1), lambda i: (i, 0)),
                   pl.BlockSpec((RB, 2), lambda i: (i, 0))],
        out_shape=[jax.ShapeDtypeStruct((NN, 1), jnp.float32),
                   jax.ShapeDtypeStruct((NN, 2), jnp.float32)],
    )(x, deg0, deg1)


def _tc_mid(p0, p1, xs, dinv, W1, b1):
    def body(p0_ref, p1_ref, xs_ref, dinv_ref, w_ref, b_ref, ha_ref, hb_ref):
        dv = dinv_ref[...]
        a = (p0_ref[...] + p1_ref[...] + xs_ref[...]) * dv
        W = w_ref[...]
        pre = a[:, 0:1] * W[0:1, :] + a[:, 1:2] * W[1:2, :] + b_ref[...]
        h = jnp.where(pre > 0, pre, jnp.expm1(pre))
        hs = h * dv
        ha_ref[...] = hs[:, 0:16]
        hb_ref[...] = hs[:, 16:32]

    return pl.pallas_call(
        body,
        grid=(GRID,),
        in_specs=[pl.BlockSpec((RB, 2), lambda i: (i, 0)),
                  pl.BlockSpec((RB, 2), lambda i: (i, 0)),
                  pl.BlockSpec((RB, 2), lambda i: (i, 0)),
                  pl.BlockSpec((RB, 1), lambda i: (i, 0)),
                  pl.BlockSpec((2, 32), lambda i: (0, 0)),
                  pl.BlockSpec((1, 32), lambda i: (0, 0))],
        out_specs=[pl.BlockSpec((RB, 16), lambda i: (i, 0)),
                   pl.BlockSpec((RB, 16), lambda i: (i, 0))],
        out_shape=[jax.ShapeDtypeStruct((NN, 16), jnp.float32),
                   jax.ShapeDtypeStruct((NN, 16), jnp.float32)],
    )(p0, p1, xs, dinv, W1, b1)


def _tc_out(g2a, g2b, ha, hb, dinv, W2, b2):
    def body(ga_ref, gb_ref, ha_ref, hb_ref, dinv_ref, w_ref, b_ref, o_ref):
        dv = dinv_ref[...]
        ua = (ga_ref[...] + ha_ref[...]) * dv
        ub = (gb_ref[...] + hb_ref[...]) * dv
        W = w_ref[...]
        o_ref[...] = (
            jnp.dot(ua, W[0:16, :], preferred_element_type=jnp.float32)
            + jnp.dot(ub, W[16:32, :], preferred_element_type=jnp.float32)
            + b_ref[...])

    return pl.pallas_call(
        body,
        grid=(GRID,),
        in_specs=[pl.BlockSpec((RB, 16), lambda i: (i, 0)),
                  pl.BlockSpec((RB, 16), lambda i: (i, 0)),
                  pl.BlockSpec((RB, 16), lambda i: (i, 0)),
                  pl.BlockSpec((RB, 16), lambda i: (i, 0)),
                  pl.BlockSpec((RB, 1), lambda i: (i, 0)),
                  pl.BlockSpec((32, 64), lambda i: (0, 0)),
                  pl.BlockSpec((1, 64), lambda i: (0, 0))],
        out_specs=pl.BlockSpec((RB, 64), lambda i: (i, 0)),
        out_shape=jax.ShapeDtypeStruct((NN, 64), jnp.float32),
    )(g2a, g2b, ha, hb, dinv, W2, b2)


def kernel(x, edge_index, W1, b1, W2, b2):
    ei = edge_index.astype(jnp.int32)
    src2d = ei[0].reshape(EROWS, LN)
    dst2d = ei[1].reshape(EROWS, LN)
    z1 = jnp.zeros((CP,), jnp.float32)
    z2 = jnp.zeros((CP, 2), jnp.float32)
    z16 = jnp.zeros((CP, 16), jnp.float32)

    deg0, deg1 = _sc_deg(dst2d, z1)
    dinv, xs = _tc_prep(x, deg0.reshape(NN, 1), deg1.reshape(NN, 1))
    p0, p1 = _sc_agg1(src2d, dst2d, xs, z2)
    ha, hb = _tc_mid(p0, p1, xs, dinv, W1, b1.reshape(1, 32))
    g2a, g2b = _sc_agg2(src2d, dst2d, ha, hb, z16)
    out = _tc_out(g2a, g2b, ha, hb, dinv, W2, b2.reshape(1, 64))
    return out


# speedup vs baseline: 13.9073x; 13.9073x over previous
import functools

import jax
import jax.numpy as jnp
from jax import lax
from jax.experimental import pallas as pl
from jax.experimental.pallas import tpu as pltpu
from jax.experimental.pallas import tpu_sc as plsc

NN = 100000
NE = 1600000
LN = 128
EROWS = NE // LN
NCORE, NSUB = 2, 16
NW = NCORE * NSUB
RB = 4000
GRID = NN // RB
CP = 6248
CPT = NN - NSUB * CP

_MESH = plsc.VectorSubcoreMesh(core_axis_name="c", subcore_axis_name="s")


def _fill(ref, n, val):
    def body(i, carry):
        ref[pl.ds(i * 16, 16)] = jnp.full((16,), val, dtype=jnp.float32)
        return carry
    lax.fori_loop(0, n // 16, body, 0)


CPF = CP // LN
CPR = CP - CPF * LN


def _zero_acc(stage, acc, s):
    base = s * CP

    def body(i, carry):
        pltpu.sync_copy(stage, acc.at[pl.ds(base + i * LN, LN)])
        return carry

    lax.fori_loop(0, CPF, body, 0)
    pltpu.sync_copy(stage.at[pl.ds(0, CPR)],
                    acc.at[pl.ds(base + CPF * LN, CPR)])

    @pl.when(s == NSUB - 1)
    def _():
        pltpu.sync_copy(stage.at[pl.ds(0, CPT)],
                        acc.at[pl.ds(NSUB * CP, CPT)])


def _copy_out(acc, stage, out, s):
    base = s * CP

    def body(i, carry):
        pltpu.sync_copy(acc.at[pl.ds(base + i * LN, LN)], stage)
        pltpu.sync_copy(stage, out.at[pl.ds(base + i * LN, LN)])
        return carry

    lax.fori_loop(0, CPF, body, 0)
    pltpu.sync_copy(acc.at[pl.ds(base + CPF * LN, CPR)],
                    stage.at[pl.ds(0, CPR)])
    pltpu.sync_copy(stage.at[pl.ds(0, CPR)],
                    out.at[pl.ds(base + CPF * LN, CPR)])

    @pl.when(s == NSUB - 1)
    def _():
        pltpu.sync_copy(acc.at[pl.ds(NSUB * CP, CPT)],
                        stage.at[pl.ds(0, CPT)])
        pltpu.sync_copy(stage.at[pl.ds(0, CPT)],
                        out.at[pl.ds(NSUB * CP, CPT)])


def _edge_range_all32(w):
    rem = EROWS - 390 * NW
    n = jnp.where(w < rem, 391, 390)
    r0 = 390 * w + jnp.minimum(w, rem)
    return r0, n


def _edge_range_16(s):
    rem = EROWS - 781 * NSUB
    n = jnp.where(s < rem, 782, 781)
    r0 = 781 * s + jnp.minimum(s, rem)
    return r0, n


@functools.partial(
    pl.kernel,
    out_type=(jax.ShapeDtypeStruct((NN,), jnp.float32),
              jax.ShapeDtypeStruct((NN,), jnp.float32)),
    mesh=_MESH,
    compiler_params=pltpu.CompilerParams(use_tc_tiling_on_sc=False),
    scratch_types=[
        pltpu.VMEM((1, LN), jnp.int32),
        pltpu.VMEM((LN,), jnp.float32),
        pltpu.VMEM_SHARED((NN,), jnp.float32),
    ],
)
def _sc_deg(dst_hbm, zeros_hbm, out0, out1, idx_d, buf, acc):
    c = lax.axis_index("c")
    s = lax.axis_index("s")
    w = c * NSUB + s
    pltpu.sync_copy(zeros_hbm, buf)
    _zero_acc(buf, acc, s)
    _fill(buf, LN, 1.0)
    plsc.subcore_barrier()

    r0, n = _edge_range_all32(w)

    def body(r, carry):
        pltpu.sync_copy(dst_hbm.at[pl.ds(r, 1)], idx_d)
        pltpu.sync_copy(buf, acc.at[idx_d.at[0]], add=True)
        return carry

    lax.fori_loop(r0, r0 + n, body, 0)
    plsc.subcore_barrier()

    @pl.when(c == 0)
    def _():
        _copy_out(acc, buf, out0, s)

    @pl.when(c == 1)
    def _():
        _copy_out(acc, buf, out1, s)


@functools.partial(
    pl.kernel,
    out_type=(jax.ShapeDtypeStruct((NN, 16), jnp.float32),
              jax.ShapeDtypeStruct((NN, 16), jnp.float32)),
    mesh=_MESH,
    compiler_params=pltpu.CompilerParams(use_tc_tiling_on_sc=False),
    scratch_types=[
        pltpu.VMEM((1, LN), jnp.int32),
        pltpu.VMEM((1, LN), jnp.int32),
        pltpu.VMEM((LN, 16), jnp.float32),
        pltpu.VMEM_SHARED((NN, 16), jnp.float32),
        pltpu.SemaphoreType.DMA,
    ],
)
def _sc_agg1(src_hbm, dst_hbm, xs_hbm, zeros_hbm, out0, out1,
             idx_s, idx_d, rows, acc, sem):
    c = lax.axis_index("c")
    s = lax.axis_index("s")
    w = c * NSUB + s
    pltpu.sync_copy(zeros_hbm, rows)
    _zero_acc(rows, acc, s)
    plsc.subcore_barrier()

    r0, n = _edge_range_all32(w)

    def body(r, carry):
        pltpu.sync_copy(src_hbm.at[pl.ds(r, 1)], idx_s)
        pltpu.sync_copy(dst_hbm.at[pl.ds(r, 1)], idx_d)
        pltpu.async_copy(xs_hbm.at[idx_s.at[0]], rows, sem).wait()
        pltpu.sync_copy(rows, acc.at[idx_d.at[0]], add=True)
        return carry

    lax.fori_loop(r0, r0 + n, body, 0)
    plsc.subcore_barrier()

    @pl.when(c == 0)
    def _():
        _copy_out(acc, rows, out0, s)

    @pl.when(c == 1)
    def _():
        _copy_out(acc, rows, out1, s)


@functools.partial(
    pl.kernel,
    out_type=(jax.ShapeDtypeStruct((NN, 16), jnp.float32),
              jax.ShapeDtypeStruct((NN, 16), jnp.float32)),
    mesh=_MESH,
    compiler_params=pltpu.CompilerParams(use_tc_tiling_on_sc=False),
    scratch_types=[
        pltpu.VMEM((1, LN), jnp.int32),
        pltpu.VMEM((1, LN), jnp.int32),
        pltpu.VMEM((LN, 16), jnp.float32),
        pltpu.VMEM_SHARED((NN, 16), jnp.float32),
        pltpu.SemaphoreType.DMA,
    ],
)
def _sc_agg2(src_hbm, dst_hbm, ha_hbm, hb_hbm, zeros_hbm, outa, outb,
             idx_s, idx_d, rows, acc, sem):
    c = lax.axis_index("c")
    s = lax.axis_index("s")
    pltpu.sync_copy(zeros_hbm, rows)
    _zero_acc(rows, acc, s)
    plsc.subcore_barrier()

    r0, n = _edge_range_16(s)

    def body(r, carry):
        pltpu.sync_copy(src_hbm.at[pl.ds(r, 1)], idx_s)
        pltpu.sync_copy(dst_hbm.at[pl.ds(r, 1)], idx_d)

        @pl.when(c == 0)
        def _():
            pltpu.async_copy(ha_hbm.at[idx_s.at[0]], rows, sem).wait()

        @pl.when(c == 1)
        def _():
            pltpu.async_copy(hb_hbm.at[idx_s.at[0]], rows, sem).wait()

        pltpu.sync_copy(rows, acc.at[idx_d.at[0]], add=True)
        return carry

    lax.fori_loop(r0, r0 + n, body, 0)
    plsc.subcore_barrier()

    @pl.when(c == 0)
    def _():
        _copy_out(acc, rows, outa, s)

    @pl.when(c == 1)
    def _():
        _copy_out(acc, rows, outb, s)


def _tc_prep(x, deg0, deg1):
    def body(x_ref, d0_ref, d1_ref, dinv_ref, xs_ref):
        deg = d0_ref[...] + d1_ref[...] + 1.0
        dinv = lax.rsqrt(deg)
        dinv_ref[...] = dinv
        xs = x_ref[...] * dinv
        xs_ref[...] = jnp.concatenate(
            [xs, jnp.zeros((xs.shape[0], 14), jnp.float32)], axis=1)

    return pl.pallas_call(
        body,
        grid=(GRID,),
        in_specs=[pl.BlockSpec((RB, 2), lambda i: (i, 0)),
                  pl.BlockSpec((RB, 1), lambda i: (i, 0)),
                  pl.BlockSpec((RB, 1), lambda i: (i, 0))],
        out_specs=[pl.BlockSpec((RB, 1), lambda i: (i, 0)),
                   pl.BlockSpec((RB, 16), lambda i: (i, 0))],
        out_shape=[jax.ShapeDtypeStruct((NN, 1), jnp.float32),
                   jax.ShapeDtypeStruct((NN, 16), jnp.float32)],
    )(x, deg0, deg1)


def _tc_mid(p0, p1, xs, dinv, W1, b1):
    def body(p0_ref, p1_ref, xs_ref, dinv_ref, w_ref, b_ref, ha_ref, hb_ref):
        dv = dinv_ref[...]
        a = (p0_ref[...][:, 0:2] + p1_ref[...][:, 0:2]
             + xs_ref[...][:, 0:2]) * dv
        W = w_ref[...]
        pre = a[:, 0:1] * W[0:1, :] + a[:, 1:2] * W[1:2, :] + b_ref[...]
        h = jnp.where(pre > 0, pre, jnp.exp(pre) - 1.0)
        hs = h * dv
        ha_ref[...] = hs[:, 0:16]
        hb_ref[...] = hs[:, 16:32]

    return pl.pallas_call(
        body,
        grid=(GRID,),
        in_specs=[pl.BlockSpec((RB, 16), lambda i: (i, 0)),
                  pl.BlockSpec((RB, 16), lambda i: (i, 0)),
                  pl.BlockSpec((RB, 16), lambda i: (i, 0)),
                  pl.BlockSpec((RB, 1), lambda i: (i, 0)),
                  pl.BlockSpec((2, 32), lambda i: (0, 0)),
                  pl.BlockSpec((1, 32), lambda i: (0, 0))],
        out_specs=[pl.BlockSpec((RB, 16), lambda i: (i, 0)),
                   pl.BlockSpec((RB, 16), lambda i: (i, 0))],
        out_shape=[jax.ShapeDtypeStruct((NN, 16), jnp.float32),
                   jax.ShapeDtypeStruct((NN, 16), jnp.float32)],
    )(p0, p1, xs, dinv, W1, b1)


def _tc_out(g2a, g2b, ha, hb, dinv, W2, b2):
    def body(ga_ref, gb_ref, ha_ref, hb_ref, dinv_ref, w_ref, b_ref, o_ref):
        dv = dinv_ref[...]
        ua = (ga_ref[...] + ha_ref[...]) * dv
        ub = (gb_ref[...] + hb_ref[...]) * dv
        W = w_ref[...]
        o_ref[...] = (
            jnp.dot(ua, W[0:16, :], preferred_element_type=jnp.float32)
            + jnp.dot(ub, W[16:32, :], preferred_element_type=jnp.float32)
            + b_ref[...])

    return pl.pallas_call(
        body,
        grid=(GRID,),
        in_specs=[pl.BlockSpec((RB, 16), lambda i: (i, 0)),
                  pl.BlockSpec((RB, 16), lambda i: (i, 0)),
                  pl.BlockSpec((RB, 16), lambda i: (i, 0)),
                  pl.BlockSpec((RB, 16), lambda i: (i, 0)),
                  pl.BlockSpec((RB, 1), lambda i: (i, 0)),
                  pl.BlockSpec((32, 64), lambda i: (0, 0)),
                  pl.BlockSpec((1, 64), lambda i: (0, 0))],
        out_specs=pl.BlockSpec((RB, 64), lambda i: (i, 0)),
        out_shape=jax.ShapeDtypeStruct((NN, 64), jnp.float32),
    )(g2a, g2b, ha, hb, dinv, W2, b2)


def kernel(x, edge_index, W1, b1, W2, b2):
    ei = edge_index.astype(jnp.int32)
    src2d = ei[0].reshape(EROWS, LN)
    dst2d = ei[1].reshape(EROWS, LN)
    z1 = jnp.zeros((LN,), jnp.float32)
    z16 = jnp.zeros((LN, 16), jnp.float32)

    deg0, deg1 = _sc_deg(dst2d, z1)
    dinv, xs = _tc_prep(x, deg0.reshape(NN, 1), deg1.reshape(NN, 1))
    p0, p1 = _sc_agg1(src2d, dst2d, xs, z16)
    ha, hb = _tc_mid(p0, p1, xs, dinv, W1, b1.reshape(1, 32))
    g2a, g2b = _sc_agg2(src2d, dst2d, ha, hb, z16)
    out = _tc_out(g2a, g2b, ha, hb, dinv, W2, b2.reshape(1, 64))
    return out


# speedup vs baseline: 33.1572x; 2.3842x over previous
import functools

import jax
import jax.numpy as jnp
from jax import lax
from jax.experimental import pallas as pl
from jax.experimental.pallas import tpu as pltpu
from jax.experimental.pallas import tpu_sc as plsc

NN = 100000
NE = 1600000
NCORE, NSUB = 2, 16
NW = NCORE * NSUB
RB = 4000
GRID = NN // RB
CP = 6248
CPT = NN - NSUB * CP

B = 400
EPW = NE // NW
EPT = NE // NSUB
NB1 = EPW // B
NB2 = EPT // B

_MESH = plsc.VectorSubcoreMesh(core_axis_name="c", subcore_axis_name="s")


def _zero_acc(stage, acc, s):
    base = s * CP
    for k in range(CP // B):
        pltpu.sync_copy(stage, acc.at[pl.ds(base + k * B, B)])
    rem = CP % B
    pltpu.sync_copy(stage.at[pl.ds(0, rem)],
                    acc.at[pl.ds(base + (CP // B) * B, rem)])

    @pl.when(s == NSUB - 1)
    def _():
        pltpu.sync_copy(stage.at[pl.ds(0, CPT)],
                        acc.at[pl.ds(NSUB * CP, CPT)])


def _copy_out(acc, stage, out, s):
    base = s * CP
    for k in range(CP // B):
        pltpu.sync_copy(acc.at[pl.ds(base + k * B, B)], stage)
        pltpu.sync_copy(stage, out.at[pl.ds(base + k * B, B)])
    rem = CP % B
    pltpu.sync_copy(acc.at[pl.ds(base + (CP // B) * B, rem)],
                    stage.at[pl.ds(0, rem)])
    pltpu.sync_copy(stage.at[pl.ds(0, rem)],
                    out.at[pl.ds(base + (CP // B) * B, rem)])

    @pl.when(s == NSUB - 1)
    def _():
        pltpu.sync_copy(acc.at[pl.ds(NSUB * CP, CPT)],
                        stage.at[pl.ds(0, CPT)])
        pltpu.sync_copy(stage.at[pl.ds(0, CPT)],
                        out.at[pl.ds(NSUB * CP, CPT)])


def _edge_pipe(src_hbm, dst_hbm, tab_hbm, acc, e0, nb,
               isA, idA, rowsA, semA, isB, idB, rowsB, semB):
    def load(iref, dref, k):
        pltpu.sync_copy(src_hbm.at[pl.ds(e0 + k * B, B)], iref.at[0])
        pltpu.sync_copy(dst_hbm.at[pl.ds(e0 + k * B, B)], dref.at[0])

    load(isA, idA, 0)
    pltpu.async_copy(tab_hbm.at[isA.at[0]], rowsA, semA)

    def pair(i, carry):
        kA = 2 * i
        load(isB, idB, kA + 1)
        pltpu.async_copy(tab_hbm.at[isB.at[0]], rowsB, semB)
        pltpu.make_async_copy(tab_hbm.at[isA.at[0]], rowsA, semA).wait()
        pltpu.sync_copy(rowsA, acc.at[idA.at[0]], add=True)

        @pl.when(kA + 2 < nb)
        def _():
            load(isA, idA, kA + 2)
            pltpu.async_copy(tab_hbm.at[isA.at[0]], rowsA, semA)

        pltpu.make_async_copy(tab_hbm.at[isB.at[0]], rowsB, semB).wait()
        pltpu.sync_copy(rowsB, acc.at[idB.at[0]], add=True)
        return carry

    lax.fori_loop(0, nb // 2, pair, 0)
    if nb % 2:
        pltpu.make_async_copy(tab_hbm.at[isA.at[0]], rowsA, semA).wait()
        pltpu.sync_copy(rowsA, acc.at[idA.at[0]], add=True)


@functools.partial(
    pl.kernel,
    out_type=(jax.ShapeDtypeStruct((NN,), jnp.float32),
              jax.ShapeDtypeStruct((NN,), jnp.float32)),
    mesh=_MESH,
    compiler_params=pltpu.CompilerParams(use_tc_tiling_on_sc=False),
    scratch_types=[
        pltpu.VMEM((1, B), jnp.int32),
        pltpu.VMEM((1, B), jnp.int32),
        pltpu.VMEM((B,), jnp.float32),
        pltpu.VMEM((B,), jnp.float32),
        pltpu.VMEM_SHARED((NN,), jnp.float32),
        pltpu.SemaphoreType.DMA,
        pltpu.SemaphoreType.DMA,
    ],
)
def _sc_deg(dst_hbm, zeros_hbm, ones_hbm, out0, out1,
            idA, idB, zbuf, ones, acc, semA, semB):
    c = lax.axis_index("c")
    s = lax.axis_index("s")
    w = c * NSUB + s
    pltpu.sync_copy(zeros_hbm, zbuf)
    pltpu.sync_copy(ones_hbm, ones)
    _zero_acc(zbuf, acc, s)
    plsc.subcore_barrier()

    e0 = w * EPW
    pltpu.async_copy(dst_hbm.at[pl.ds(e0, B)], idA.at[0], semA)

    def pair(i, carry):
        kA = 2 * i
        pltpu.async_copy(dst_hbm.at[pl.ds(e0 + (kA + 1) * B, B)],
                         idB.at[0], semB)
        pltpu.make_async_copy(dst_hbm.at[pl.ds(e0, B)],
                              idA.at[0], semA).wait()
        pltpu.sync_copy(ones, acc.at[idA.at[0]], add=True)

        @pl.when(kA + 2 < NB1)
        def _():
            pltpu.async_copy(dst_hbm.at[pl.ds(e0 + (kA + 2) * B, B)],
                             idA.at[0], semA)

        pltpu.make_async_copy(dst_hbm.at[pl.ds(e0, B)],
                              idB.at[0], semB).wait()
        pltpu.sync_copy(ones, acc.at[idB.at[0]], add=True)
        return carry

    lax.fori_loop(0, NB1 // 2, pair, 0)
    if NB1 % 2:
        pltpu.make_async_copy(dst_hbm.at[pl.ds(e0, B)],
                              idA.at[0], semA).wait()
        pltpu.sync_copy(ones, acc.at[idA.at[0]], add=True)
    plsc.subcore_barrier()

    @pl.when(c == 0)
    def _():
        _copy_out(acc, zbuf, out0, s)

    @pl.when(c == 1)
    def _():
        _copy_out(acc, zbuf, out1, s)


@functools.partial(
    pl.kernel,
    out_type=(jax.ShapeDtypeStruct((NN, 16), jnp.float32),
              jax.ShapeDtypeStruct((NN, 16), jnp.float32)),
    mesh=_MESH,
    compiler_params=pltpu.CompilerParams(use_tc_tiling_on_sc=False),
    scratch_types=[
        pltpu.VMEM((1, B), jnp.int32),
        pltpu.VMEM((1, B), jnp.int32),
        pltpu.VMEM((1, B), jnp.int32),
        pltpu.VMEM((1, B), jnp.int32),
        pltpu.VMEM((B, 16), jnp.float32),
        pltpu.VMEM((B, 16), jnp.float32),
        pltpu.VMEM_SHARED((NN, 16), jnp.float32),
        pltpu.SemaphoreType.DMA,
        pltpu.SemaphoreType.DMA,
    ],
)
def _sc_agg1(src_hbm, dst_hbm, xs_hbm, zeros_hbm, out0, out1,
             isA, idA, isB, idB, rowsA, rowsB, acc, semA, semB):
    c = lax.axis_index("c")
    s = lax.axis_index("s")
    w = c * NSUB + s
    pltpu.sync_copy(zeros_hbm, rowsA)
    _zero_acc(rowsA, acc, s)
    plsc.subcore_barrier()

    _edge_pipe(src_hbm, dst_hbm, xs_hbm, acc, w * EPW, NB1,
               isA, idA, rowsA, semA, isB, idB, rowsB, semB)
    plsc.subcore_barrier()

    @pl.when(c == 0)
    def _():
        _copy_out(acc, rowsA, out0, s)

    @pl.when(c == 1)
    def _():
        _copy_out(acc, rowsA, out1, s)


@functools.partial(
    pl.kernel,
    out_type=(jax.ShapeDtypeStruct((NN, 16), jnp.float32),
              jax.ShapeDtypeStruct((NN, 16), jnp.float32)),
    mesh=_MESH,
    compiler_params=pltpu.CompilerParams(use_tc_tiling_on_sc=False),
    scratch_types=[
        pltpu.VMEM((1, B), jnp.int32),
        pltpu.VMEM((1, B), jnp.int32),
        pltpu.VMEM((1, B), jnp.int32),
        pltpu.VMEM((1, B), jnp.int32),
        pltpu.VMEM((B, 16), jnp.float32),
        pltpu.VMEM((B, 16), jnp.float32),
        pltpu.VMEM_SHARED((NN, 16), jnp.float32),
        pltpu.SemaphoreType.DMA,
        pltpu.SemaphoreType.DMA,
    ],
)
def _sc_agg2(src_hbm, dst_hbm, ha_hbm, hb_hbm, zeros_hbm, outa, outb,
             isA, idA, isB, idB, rowsA, rowsB, acc, semA, semB):
    c = lax.axis_index("c")
    s = lax.axis_index("s")
    pltpu.sync_copy(zeros_hbm, rowsA)
    _zero_acc(rowsA, acc, s)
    plsc.subcore_barrier()

    e0 = s * EPT

    @pl.when(c == 0)
    def _():
        _edge_pipe(src_hbm, dst_hbm, ha_hbm, acc, e0, NB2,
                   isA, idA, rowsA, semA, isB, idB, rowsB, semB)

    @pl.when(c == 1)
    def _():
        _edge_pipe(src_hbm, dst_hbm, hb_hbm, acc, e0, NB2,
                   isA, idA, rowsA, semA, isB, idB, rowsB, semB)

    plsc.subcore_barrier()

    @pl.when(c == 0)
    def _():
        _copy_out(acc, rowsA, outa, s)

    @pl.when(c == 1)
    def _():
        _copy_out(acc, rowsA, outb, s)


def _tc_prep(x, deg0, deg1):
    def body(x_ref, d0_ref, d1_ref, dinv_ref, xs_ref):
        deg = d0_ref[...] + d1_ref[...] + 1.0
        dinv = lax.rsqrt(deg)
        dinv_ref[...] = dinv
        xs = x_ref[...] * dinv
        xs_ref[...] = jnp.concatenate(
            [xs, jnp.zeros((xs.shape[0], 14), jnp.float32)], axis=1)

    return pl.pallas_call(
        body,
        grid=(GRID,),
        in_specs=[pl.BlockSpec((RB, 2), lambda i: (i, 0)),
                  pl.BlockSpec((RB, 1), lambda i: (i, 0)),
                  pl.BlockSpec((RB, 1), lambda i: (i, 0))],
        out_specs=[pl.BlockSpec((RB, 1), lambda i: (i, 0)),
                   pl.BlockSpec((RB, 16), lambda i: (i, 0))],
        out_shape=[jax.ShapeDtypeStruct((NN, 1), jnp.float32),
                   jax.ShapeDtypeStruct((NN, 16), jnp.float32)],
    )(x, deg0, deg1)


def _tc_mid(p0, p1, xs, dinv, W1, b1):
    def body(p0_ref, p1_ref, xs_ref, dinv_ref, w_ref, b_ref, ha_ref, hb_ref):
        dv = dinv_ref[...]
        a = (p0_ref[...][:, 0:2] + p1_ref[...][:, 0:2]
             + xs_ref[...][:, 0:2]) * dv
        W = w_ref[...]
        pre = a[:, 0:1] * W[0:1, :] + a[:, 1:2] * W[1:2, :] + b_ref[...]
        h = jnp.where(pre > 0, pre, jnp.exp(pre) - 1.0)
        hs = h * dv
        ha_ref[...] = hs[:, 0:16]
        hb_ref[...] = hs[:, 16:32]

    return pl.pallas_call(
        body,
        grid=(GRID,),
        in_specs=[pl.BlockSpec((RB, 16), lambda i: (i, 0)),
                  pl.BlockSpec((RB, 16), lambda i: (i, 0)),
                  pl.BlockSpec((RB, 16), lambda i: (i, 0)),
                  pl.BlockSpec((RB, 1), lambda i: (i, 0)),
                  pl.BlockSpec((2, 32), lambda i: (0, 0)),
                  pl.BlockSpec((1, 32), lambda i: (0, 0))],
        out_specs=[pl.BlockSpec((RB, 16), lambda i: (i, 0)),
                   pl.BlockSpec((RB, 16), lambda i: (i, 0))],
        out_shape=[jax.ShapeDtypeStruct((NN, 16), jnp.float32),
                   jax.ShapeDtypeStruct((NN, 16), jnp.float32)],
    )(p0, p1, xs, dinv, W1, b1)


def _tc_out(g2a, g2b, ha, hb, dinv, W2, b2):
    def body(ga_ref, gb_ref, ha_ref, hb_ref, dinv_ref, w_ref, b_ref, o_ref):
        dv = dinv_ref[...]
        ua = (ga_ref[...] + ha_ref[...]) * dv
        ub = (gb_ref[...] + hb_ref[...]) * dv
        W = w_ref[...]
        o_ref[...] = (
            jnp.dot(ua, W[0:16, :], preferred_element_type=jnp.float32)
            + jnp.dot(ub, W[16:32, :], preferred_element_type=jnp.float32)
            + b_ref[...])

    return pl.pallas_call(
        body,
        grid=(GRID,),
        in_specs=[pl.BlockSpec((RB, 16), lambda i: (i, 0)),
                  pl.BlockSpec((RB, 16), lambda i: (i, 0)),
                  pl.BlockSpec((RB, 16), lambda i: (i, 0)),
                  pl.BlockSpec((RB, 16), lambda i: (i, 0)),
                  pl.BlockSpec((RB, 1), lambda i: (i, 0)),
                  pl.BlockSpec((32, 64), lambda i: (0, 0)),
                  pl.BlockSpec((1, 64), lambda i: (0, 0))],
        out_specs=pl.BlockSpec((RB, 64), lambda i: (i, 0)),
        out_shape=jax.ShapeDtypeStruct((NN, 64), jnp.float32),
    )(g2a, g2b, ha, hb, dinv, W2, b2)


def kernel(x, edge_index, W1, b1, W2, b2):
    ei = edge_index.astype(jnp.int32)
    src = ei[0]
    dst = ei[1]
    z1 = jnp.zeros((B,), jnp.float32)
    o1 = jnp.ones((B,), jnp.float32)
    z16 = jnp.zeros((B, 16), jnp.float32)

    deg0, deg1 = _sc_deg(dst, z1, o1)
    dinv, xs = _tc_prep(x, deg0.reshape(NN, 1), deg1.reshape(NN, 1))
    p0, p1 = _sc_agg1(src, dst, xs, z16)
    ha, hb = _tc_mid(p0, p1, xs, dinv, W1, b1.reshape(1, 32))
    g2a, g2b = _sc_agg2(src, dst, ha, hb, z16)
    out = _tc_out(g2a, g2b, ha, hb, dinv, W2, b2.reshape(1, 64))
    return out


# speedup vs baseline: 37.7692x; 1.1391x over previous
import functools

import jax
import jax.numpy as jnp
from jax import lax
from jax.experimental import pallas as pl
from jax.experimental.pallas import tpu as pltpu
from jax.experimental.pallas import tpu_sc as plsc

NN = 100000
NE = 1600000
NCORE, NSUB = 2, 16
NW = NCORE * NSUB
RB = 4000
GRID = NN // RB
CP = 6248
CPT = NN - NSUB * CP

B1 = 400
B2 = 800
BD = 5000
EPW = NE // NW
EPT = NE // NSUB
NB1 = EPW // B1
NB2 = EPT // B2
NBD = EPW // BD

_MESH = plsc.VectorSubcoreMesh(core_axis_name="c", subcore_axis_name="s")


def _zero_acc(stage, acc, s, b):
    base = s * CP
    for k in range(CP // b):
        pltpu.sync_copy(stage, acc.at[pl.ds(base + k * b, b)])
    rem = CP % b
    pltpu.sync_copy(stage.at[pl.ds(0, rem)],
                    acc.at[pl.ds(base + (CP // b) * b, rem)])

    @pl.when(s == NSUB - 1)
    def _():
        pltpu.sync_copy(stage.at[pl.ds(0, CPT)],
                        acc.at[pl.ds(NSUB * CP, CPT)])


def _copy_out(acc, stA, stB, out, s, b, semA, semB):
    base = s * CP
    nfull = CP // b
    rem = CP % b
    sts = (stA, stB)
    sems = (semA, semB)
    for k in range(nfull):
        st, sem = sts[k % 2], sems[k % 2]
        if k >= 2:
            pltpu.make_async_copy(st, out.at[pl.ds(base + (k - 2) * b, b)],
                                  sem).wait()
        pltpu.sync_copy(acc.at[pl.ds(base + k * b, b)], st)
        pltpu.async_copy(st, out.at[pl.ds(base + k * b, b)], sem)
    for k in range(max(0, nfull - 2), nfull):
        st, sem = sts[k % 2], sems[k % 2]
        pltpu.make_async_copy(st, out.at[pl.ds(base + k * b, b)], sem).wait()
    if rem:
        pltpu.sync_copy(acc.at[pl.ds(base + nfull * b, rem)],
                        stA.at[pl.ds(0, rem)])
        pltpu.sync_copy(stA.at[pl.ds(0, rem)],
                        out.at[pl.ds(base + nfull * b, rem)])

    @pl.when(s == NSUB - 1)
    def _():
        pltpu.sync_copy(acc.at[pl.ds(NSUB * CP, CPT)],
                        stA.at[pl.ds(0, CPT)])
        pltpu.sync_copy(stA.at[pl.ds(0, CPT)],
                        out.at[pl.ds(NSUB * CP, CPT)])


def _edge_pipe(src_hbm, dst_hbm, tab_hbm, acc, e0, b, nb,
               isA, idA, rowsA, semA, isB, idB, rowsB, semB):
    def load(iref, dref, k):
        pltpu.sync_copy(src_hbm.at[pl.ds(e0 + k * b, b)], iref.at[0])
        pltpu.sync_copy(dst_hbm.at[pl.ds(e0 + k * b, b)], dref.at[0])

    load(isA, idA, 0)
    pltpu.async_copy(tab_hbm.at[isA.at[0]], rowsA, semA)

    def pair(i, carry):
        kA = 2 * i
        load(isB, idB, kA + 1)
        pltpu.async_copy(tab_hbm.at[isB.at[0]], rowsB, semB)
        pltpu.make_async_copy(tab_hbm.at[isA.at[0]], rowsA, semA).wait()
        pltpu.sync_copy(rowsA, acc.at[idA.at[0]], add=True)

        @pl.when(kA + 2 < nb)
        def _():
            load(isA, idA, kA + 2)
            pltpu.async_copy(tab_hbm.at[isA.at[0]], rowsA, semA)

        pltpu.make_async_copy(tab_hbm.at[isB.at[0]], rowsB, semB).wait()
        pltpu.sync_copy(rowsB, acc.at[idB.at[0]], add=True)
        return carry

    lax.fori_loop(0, nb // 2, pair, 0)
    if nb % 2:
        pltpu.make_async_copy(tab_hbm.at[isA.at[0]], rowsA, semA).wait()
        pltpu.sync_copy(rowsA, acc.at[idA.at[0]], add=True)


@functools.partial(
    pl.kernel,
    out_type=(jax.ShapeDtypeStruct((NN,), jnp.float32),
              jax.ShapeDtypeStruct((NN,), jnp.float32)),
    mesh=_MESH,
    compiler_params=pltpu.CompilerParams(use_tc_tiling_on_sc=False),
    scratch_types=[
        pltpu.VMEM((1, BD), jnp.int32),
        pltpu.VMEM((1, BD), jnp.int32),
        pltpu.VMEM((BD,), jnp.float32),
        pltpu.VMEM((BD,), jnp.float32),
        pltpu.VMEM_SHARED((NN,), jnp.float32),
        pltpu.SemaphoreType.DMA,
        pltpu.SemaphoreType.DMA,
    ],
)
def _sc_deg(dst_hbm, zeros_hbm, ones_hbm, out0, out1,
            idA, idB, zbuf, ones, acc, semA, semB):
    c = lax.axis_index("c")
    s = lax.axis_index("s")
    w = c * NSUB + s
    pltpu.sync_copy(zeros_hbm, zbuf)
    pltpu.sync_copy(ones_hbm, ones)
    _zero_acc(zbuf, acc, s, BD)
    plsc.subcore_barrier()

    e0 = w * EPW
    pltpu.async_copy(dst_hbm.at[pl.ds(e0, BD)], idA.at[0], semA)

    def pair(i, carry):
        kA = 2 * i
        pltpu.async_copy(dst_hbm.at[pl.ds(e0 + (kA + 1) * BD, BD)],
                         idB.at[0], semB)
        pltpu.make_async_copy(dst_hbm.at[pl.ds(e0, BD)],
                              idA.at[0], semA).wait()
        pltpu.sync_copy(ones, acc.at[idA.at[0]], add=True)

        @pl.when(kA + 2 < NBD)
        def _():
            pltpu.async_copy(dst_hbm.at[pl.ds(e0 + (kA + 2) * BD, BD)],
                             idA.at[0], semA)

        pltpu.make_async_copy(dst_hbm.at[pl.ds(e0, BD)],
                              idB.at[0], semB).wait()
        pltpu.sync_copy(ones, acc.at[idB.at[0]], add=True)
        return carry

    lax.fori_loop(0, NBD // 2, pair, 0)
    if NBD % 2:
        pltpu.make_async_copy(dst_hbm.at[pl.ds(e0, BD)],
                              idA.at[0], semA).wait()
        pltpu.sync_copy(ones, acc.at[idA.at[0]], add=True)
    plsc.subcore_barrier()

    @pl.when(c == 0)
    def _():
        _copy_out(acc, zbuf, ones, out0, s, BD, semA, semB)

    @pl.when(c == 1)
    def _():
        _copy_out(acc, zbuf, ones, out1, s, BD, semA, semB)


@functools.partial(
    pl.kernel,
    out_type=(jax.ShapeDtypeStruct((NN, 16), jnp.float32),
              jax.ShapeDtypeStruct((NN, 16), jnp.float32)),
    mesh=_MESH,
    compiler_params=pltpu.CompilerParams(use_tc_tiling_on_sc=False),
    scratch_types=[
        pltpu.VMEM((1, B1), jnp.int32),
        pltpu.VMEM((1, B1), jnp.int32),
        pltpu.VMEM((1, B1), jnp.int32),
        pltpu.VMEM((1, B1), jnp.int32),
        pltpu.VMEM((B1, 16), jnp.float32),
        pltpu.VMEM((B1, 16), jnp.float32),
        pltpu.VMEM_SHARED((NN, 16), jnp.float32),
        pltpu.SemaphoreType.DMA,
        pltpu.SemaphoreType.DMA,
    ],
)
def _sc_agg1(src_hbm, dst_hbm, xs_hbm, zeros_hbm, out0, out1,
             isA, idA, isB, idB, rowsA, rowsB, acc, semA, semB):
    c = lax.axis_index("c")
    s = lax.axis_index("s")
    w = c * NSUB + s
    pltpu.sync_copy(zeros_hbm, rowsA)
    _zero_acc(rowsA, acc, s, B1)
    plsc.subcore_barrier()

    _edge_pipe(src_hbm, dst_hbm, xs_hbm, acc, w * EPW, B1, NB1,
               isA, idA, rowsA, semA, isB, idB, rowsB, semB)
    plsc.subcore_barrier()

    @pl.when(c == 0)
    def _():
        _copy_out(acc, rowsA, rowsB, out0, s, B1, semA, semB)

    @pl.when(c == 1)
    def _():
        _copy_out(acc, rowsA, rowsB, out1, s, B1, semA, semB)


@functools.partial(
    pl.kernel,
    out_type=(jax.ShapeDtypeStruct((NN, 16), jnp.float32),
              jax.ShapeDtypeStruct((NN, 16), jnp.float32)),
    mesh=_MESH,
    compiler_params=pltpu.CompilerParams(use_tc_tiling_on_sc=False),
    scratch_types=[
        pltpu.VMEM((1, B2), jnp.int32),
        pltpu.VMEM((1, B2), jnp.int32),
        pltpu.VMEM((1, B2), jnp.int32),
        pltpu.VMEM((1, B2), jnp.int32),
        pltpu.VMEM((B2, 16), jnp.float32),
        pltpu.VMEM((B2, 16), jnp.float32),
        pltpu.VMEM_SHARED((NN, 16), jnp.float32),
        pltpu.SemaphoreType.DMA,
        pltpu.SemaphoreType.DMA,
    ],
)
def _sc_agg2(src_hbm, dst_hbm, ha_hbm, hb_hbm, zeros_hbm, outa, outb,
             isA, idA, isB, idB, rowsA, rowsB, acc, semA, semB):
    c = lax.axis_index("c")
    s = lax.axis_index("s")
    pltpu.sync_copy(zeros_hbm, rowsA)
    _zero_acc(rowsA, acc, s, B2)
    plsc.subcore_barrier()

    e0 = s * EPT

    @pl.when(c == 0)
    def _():
        _edge_pipe(src_hbm, dst_hbm, ha_hbm, acc, e0, B2, NB2,
                   isA, idA, rowsA, semA, isB, idB, rowsB, semB)

    @pl.when(c == 1)
    def _():
        _edge_pipe(src_hbm, dst_hbm, hb_hbm, acc, e0, B2, NB2,
                   isA, idA, rowsA, semA, isB, idB, rowsB, semB)

    plsc.subcore_barrier()

    @pl.when(c == 0)
    def _():
        _copy_out(acc, rowsA, rowsB, outa, s, B2, semA, semB)

    @pl.when(c == 1)
    def _():
        _copy_out(acc, rowsA, rowsB, outb, s, B2, semA, semB)


def _tc_prep(x, deg0, deg1):
    def body(x_ref, d0_ref, d1_ref, dinv_ref, xs_ref):
        deg = d0_ref[...] + d1_ref[...] + 1.0
        dinv = lax.rsqrt(deg)
        dinv_ref[...] = dinv
        xs = x_ref[...] * dinv
        xs_ref[...] = jnp.concatenate(
            [xs, jnp.zeros((xs.shape[0], 14), jnp.float32)], axis=1)

    return pl.pallas_call(
        body,
        grid=(GRID,),
        in_specs=[pl.BlockSpec((RB, 2), lambda i: (i, 0)),
                  pl.BlockSpec((RB, 1), lambda i: (i, 0)),
                  pl.BlockSpec((RB, 1), lambda i: (i, 0))],
        out_specs=[pl.BlockSpec((RB, 1), lambda i: (i, 0)),
                   pl.BlockSpec((RB, 16), lambda i: (i, 0))],
        out_shape=[jax.ShapeDtypeStruct((NN, 1), jnp.float32),
                   jax.ShapeDtypeStruct((NN, 16), jnp.float32)],
    )(x, deg0, deg1)


def _tc_mid(p0, p1, xs, dinv, W1, b1):
    def body(p0_ref, p1_ref, xs_ref, dinv_ref, w_ref, b_ref, ha_ref, hb_ref):
        dv = dinv_ref[...]
        a = (p0_ref[...][:, 0:2] + p1_ref[...][:, 0:2]
             + xs_ref[...][:, 0:2]) * dv
        W = w_ref[...]
        pre = a[:, 0:1] * W[0:1, :] + a[:, 1:2] * W[1:2, :] + b_ref[...]
        h = jnp.where(pre > 0, pre, jnp.exp(pre) - 1.0)
        hs = h * dv
        ha_ref[...] = hs[:, 0:16]
        hb_ref[...] = hs[:, 16:32]

    return pl.pallas_call(
        body,
        grid=(GRID,),
        in_specs=[pl.BlockSpec((RB, 16), lambda i: (i, 0)),
                  pl.BlockSpec((RB, 16), lambda i: (i, 0)),
                  pl.BlockSpec((RB, 16), lambda i: (i, 0)),
                  pl.BlockSpec((RB, 1), lambda i: (i, 0)),
                  pl.BlockSpec((2, 32), lambda i: (0, 0)),
                  pl.BlockSpec((1, 32), lambda i: (0, 0))],
        out_specs=[pl.BlockSpec((RB, 16), lambda i: (i, 0)),
                   pl.BlockSpec((RB, 16), lambda i: (i, 0))],
        out_shape=[jax.ShapeDtypeStruct((NN, 16), jnp.float32),
                   jax.ShapeDtypeStruct((NN, 16), jnp.float32)],
    )(p0, p1, xs, dinv, W1, b1)


def _tc_out(g2a, g2b, ha, hb, dinv, W2, b2):
    def body(ga_ref, gb_ref, ha_ref, hb_ref, dinv_ref, w_ref, b_ref, o_ref):
        dv = dinv_ref[...]
        ua = (ga_ref[...] + ha_ref[...]) * dv
        ub = (gb_ref[...] + hb_ref[...]) * dv
        W = w_ref[...]
        o_ref[...] = (
            jnp.dot(ua, W[0:16, :], preferred_element_type=jnp.float32)
            + jnp.dot(ub, W[16:32, :], preferred_element_type=jnp.float32)
            + b_ref[...])

    return pl.pallas_call(
        body,
        grid=(GRID,),
        in_specs=[pl.BlockSpec((RB, 16), lambda i: (i, 0)),
                  pl.BlockSpec((RB, 16), lambda i: (i, 0)),
                  pl.BlockSpec((RB, 16), lambda i: (i, 0)),
                  pl.BlockSpec((RB, 16), lambda i: (i, 0)),
                  pl.BlockSpec((RB, 1), lambda i: (i, 0)),
                  pl.BlockSpec((32, 64), lambda i: (0, 0)),
                  pl.BlockSpec((1, 64), lambda i: (0, 0))],
        out_specs=pl.BlockSpec((RB, 64), lambda i: (i, 0)),
        out_shape=jax.ShapeDtypeStruct((NN, 64), jnp.float32),
    )(g2a, g2b, ha, hb, dinv, W2, b2)


def kernel(x, edge_index, W1, b1, W2, b2):
    ei = edge_index.astype(jnp.int32)
    src = ei[0]
    dst = ei[1]
    zd = jnp.zeros((BD,), jnp.float32)
    od = jnp.ones((BD,), jnp.float32)
    z16a = jnp.zeros((B1, 16), jnp.float32)
    z16b = jnp.zeros((B2, 16), jnp.float32)

    deg0, deg1 = _sc_deg(dst, zd, od)
    dinv, xs = _tc_prep(x, deg0.reshape(NN, 1), deg1.reshape(NN, 1))
    p0, p1 = _sc_agg1(src, dst, xs, z16a)
    ha, hb = _tc_mid(p0, p1, xs, dinv, W1, b1.reshape(1, 32))
    g2a, g2b = _sc_agg2(src, dst, ha, hb, z16b)
    out = _tc_out(g2a, g2b, ha, hb, dinv, W2, b2.reshape(1, 64))
    return out
